# baseline (speedup 1.0000x reference)
import math

import jax
import jax.numpy as jnp
from jax import lax
from jax.experimental import pallas as pl
from jax.experimental.pallas import tpu as pltpu
from jax.experimental.pallas import tpu_sc as plsc

F = 26
FD = 38461
FN = F * FD
EMB = 16
HID = 64
B = 4096
EPS = 1e-5

NC = 2
NS = 16
FPC = F // NC
LANE = 128
KCH = 2
NJ = FPC * KCH
CTAB = FPC * FD
G2 = 3
R2 = (FPC + G2 - 1) // G2
DINV_S = 1.0 / math.sqrt(27.0)
BNI = 1.0 / math.sqrt(1.0 + EPS)

_MESH = plsc.VectorSubcoreMesh(core_axis_name="c", subcore_axis_name="s")


_RPB = 13312


def _repack_body(wt_ref, bt_ref, wlin_ref, blin_ref):
  blk = wt_ref[...]
  wlin_ref[...] = jnp.transpose(blk, (1, 0))
  blin_ref[...] = bt_ref[0]


_RPG = (FN + _RPB - 1) // _RPB
_WLR = _RPG * _RPB

_repack = pl.pallas_call(
    _repack_body,
    grid=(_RPG,),
    in_specs=[pl.BlockSpec((EMB, _RPB), lambda i: (0, i)),
              pl.BlockSpec((1, _RPB), lambda i: (0, i))],
    out_specs=[pl.BlockSpec((_RPB, EMB), lambda i: (i, 0)),
               pl.BlockSpec((_RPB,), lambda i: (i,))],
    out_shape=[jax.ShapeDtypeStruct((_WLR, EMB), jnp.float32),
               jax.ShapeDtypeStruct((FN,), jnp.float32)],
)


def _sc1_body(xw_h, w_h, b_h,
              e_o, cnt_o, bg_o,
              xbuf, iloc, iglob, zb, ob, cbuf, bbuf, ebuf, ctab,
              semw, semb, semc):
  c = lax.axis_index("c")
  s = lax.axis_index("s")
  w2 = w_h
  for i in range(LANE // 16):
    zb[pl.ds(i * 16, 16)] = jnp.zeros((16,), jnp.float32)
    ob[pl.ds(i * 16, 16)] = jnp.full((16,), 1.0, jnp.float32)
  pltpu.sync_copy(xw_h.at[c, s], xbuf)
  fbase = (c * FPC) * FD

  @pl.loop(0, NJ)
  def _idx(j):
    jf = j // KCH
    og = fbase + jf * FD
    ol = jf * FD

    @pl.loop(0, LANE, step=16)
    def _v(i):
      v = xbuf[j, pl.ds(i, 16)]
      iglob[j, pl.ds(i, 16)] = v + og
      iloc[j, pl.ds(i, 16)] = v + ol

  wd = [pltpu.async_copy(w2.at[iglob.at[j]], ebuf.at[j], semw)
        for j in range(NJ)]
  bd = [pltpu.async_copy(b_h.at[iglob.at[j]], bbuf.at[j], semb)
        for j in range(NJ)]

  zd = [pltpu.async_copy(zb, ctab.at[iloc.at[j]], semc) for j in range(NJ)]
  for d in zd:
    d.wait()
  plsc.subcore_barrier()
  ad = [pltpu.async_copy(ob, ctab.at[iloc.at[j]], semc, add=True)
        for j in range(NJ)]
  for d in ad:
    d.wait()
  plsc.subcore_barrier()
  cd = [pltpu.async_copy(ctab.at[iloc.at[j]], cbuf.at[j], semc)
        for j in range(NJ)]
  for d in cd:
    d.wait()
  pltpu.sync_copy(cbuf, cnt_o.at[c, s])
  for d in bd:
    d.wait()
  pltpu.sync_copy(bbuf, bg_o.at[c, s])
  for d in wd:
    d.wait()
  pltpu.sync_copy(ebuf, e_o.at[c, s])


_sc1 = pl.kernel(
    _sc1_body,
    out_type=(
        jax.ShapeDtypeStruct((NC, NS, NJ, LANE, EMB), jnp.float32),
        jax.ShapeDtypeStruct((NC, NS, NJ, LANE), jnp.float32),
        jax.ShapeDtypeStruct((NC, NS, NJ, LANE), jnp.float32),
    ),
    mesh=_MESH,
    scratch_types=[
        pltpu.VMEM((NJ, LANE), jnp.int32),
        pltpu.VMEM((NJ, LANE), jnp.int32),
        pltpu.VMEM((NJ, LANE), jnp.int32),
        pltpu.VMEM((LANE,), jnp.float32),
        pltpu.VMEM((LANE,), jnp.float32),
        pltpu.VMEM((NJ, LANE), jnp.float32),
        pltpu.VMEM((NJ, LANE), jnp.float32),
        pltpu.VMEM((NJ, LANE, EMB), jnp.float32),
        pltpu.VMEM_SHARED((CTAB,), jnp.float32),
        pltpu.SemaphoreType.DMA,
        pltpu.SemaphoreType.DMA,
        pltpu.SemaphoreType.DMA,
    ],
    compiler_params=pltpu.CompilerParams(use_tc_tiling_on_sc=False),
)


def _sc2_body(xw_h, m_h, z2_h,
              agg_o,
              iloc2, mbuf, zrow, gbuf, wtab, sem, semo):
  c = lax.axis_index("c")
  s = lax.axis_index("s")
  pltpu.sync_copy(z2_h, zrow)
  pltpu.sync_copy(xw_h.at[c, s], iloc2)
  pltpu.sync_copy(m_h.at[pl.ds(s * KCH, KCH)], mbuf)

  @pl.loop(0, NJ)
  def _idx(j):
    jf = j // KCH
    ol = (jf % G2) * FD

    @pl.loop(0, LANE, step=16)
    def _v(i):
      iloc2[j, pl.ds(i, 16)] = iloc2[j, pl.ds(i, 16)] + ol

  slot_wr = [None, None]
  for r in range(R2):
    nf = min(G2, FPC - r * G2)
    lo = r * G2 * KCH
    n = nf * KCH
    zd = [pltpu.async_copy(zrow, wtab.at[iloc2.at[lo + t]], sem)
          for t in range(n)]
    for d in zd:
      d.wait()
    plsc.subcore_barrier()
    ad = [pltpu.async_copy(mbuf.at[(lo + t) % KCH], wtab.at[iloc2.at[lo + t]],
                           sem, add=True) for t in range(n)]
    for d in ad:
      d.wait()
    plsc.subcore_barrier()
    for t in range(n):
      sl = t % 2
      if slot_wr[sl] is not None:
        slot_wr[sl].wait()
      pltpu.async_copy(wtab.at[iloc2.at[lo + t]], gbuf.at[sl], sem).wait()
      slot_wr[sl] = pltpu.async_copy(gbuf.at[sl], agg_o.at[c, s, lo + t], semo)
    plsc.subcore_barrier()
  for d in slot_wr:
    if d is not None:
      d.wait()


_sc2 = pl.kernel(
    _sc2_body,
    out_type=jax.ShapeDtypeStruct((NC, NS, NJ, LANE, EMB), jnp.float32),
    mesh=_MESH,
    scratch_types=[
        pltpu.VMEM((NJ, LANE), jnp.int32),
        pltpu.VMEM((KCH, LANE, EMB), jnp.float32),
        pltpu.VMEM((LANE, EMB), jnp.float32),
        pltpu.VMEM((2, LANE, EMB), jnp.float32),
        pltpu.VMEM_SHARED((G2 * FD, EMB), jnp.float32),
        pltpu.SemaphoreType.DMA,
        pltpu.SemaphoreType.DMA,
    ],
    compiler_params=pltpu.CompilerParams(use_tc_tiling_on_sc=False),
)


def _tc1_body(e_ref, cnt_ref, gw12_ref, gb12r_ref, m_ref):
  gw12 = gw12_ref[...]
  gb12r = gb12r_ref[...]
  for k in range(KCH):
    acc = jnp.zeros((LANE, EMB), jnp.float32)
    for c in range(NC):
      for jf in range(FPC):
        j = jf * KCH + k
        df = lax.rsqrt(cnt_ref[c, 0, j] + 1.0)
        acc = acc + df[:, None] * e_ref[c, 0, j]
    m = jnp.dot(acc, gw12, preferred_element_type=jnp.float32)
    m_ref[0, k] = DINV_S * m + gb12r


def _tc2_body(e_ref, agg_ref, cnt_ref, bg_ref,
              gw12_ref, gb12r_ref, gb2r_ref,
              lw1_ref, lb1r_ref, lg1r_ref, lbe1r_ref,
              lw2_ref, lb2r_ref, lg2r_ref, lbe2r_ref,
              lw3_ref, lb3r_ref, out_ref):
  gw12 = gw12_ref[...]
  gb12r = gb12r_ref[...]
  gb2r = gb2r_ref[...]
  for k in range(KCH):
    hsum = jnp.zeros((LANE, EMB), jnp.float32)
    esum = jnp.zeros((LANE, EMB), jnp.float32)
    ssq = jnp.zeros((LANE, EMB), jnp.float32)
    bsum = jnp.zeros((LANE,), jnp.float32)
    for c in range(NC):
      for jf in range(FPC):
        j = jf * KCH + k
        ef = e_ref[c, 0, j]
        af = agg_ref[c, 0, j]
        df = lax.rsqrt(cnt_ref[c, 0, j] + 1.0)[:, None]
        d2 = df * df
        egf = jnp.dot(ef, gw12, preferred_element_type=jnp.float32)
        vf = (d2 * d2) * egf + d2 * gb12r + gb2r
        xf = (DINV_S * df) * af + vf
        hsum = hsum + xf
        esum = esum + ef
        ssq = ssq + ef * xf
        bsum = bsum + bg_ref[c, 0, j]
    fm = 0.5 * jnp.sum(esum * hsum - ssq, axis=1)
    z = jnp.dot(hsum, lw1_ref[...], preferred_element_type=jnp.float32)
    z = jnp.maximum((z + lb1r_ref[...]) * BNI * lg1r_ref[...] + lbe1r_ref[...], 0.0)
    z = jnp.dot(z, lw2_ref[...], preferred_element_type=jnp.float32)
    z = jnp.maximum((z + lb2r_ref[...]) * BNI * lg2r_ref[...] + lbe2r_ref[...], 0.0)
    z = jnp.dot(z, lw3_ref[...], preferred_element_type=jnp.float32) + lb3r_ref[...]
    out_ref[0, k] = bsum + fm + z[:, 0]


def _const_spec(shape):
  return pl.BlockSpec(shape, lambda s: tuple(0 for _ in shape))


def kernel(x, field_mask, new_field_mask, known_mask, b_param, w_param,
           gw1, gb1, gw2, gb2, lw1, lb1, lg1, lbe1, lw2, lb2, lg2, lbe2,
           lw3, lb3):
  x = x.astype(jnp.int32)
  xw = (x.T.reshape(NC, FPC, NS, KCH, LANE)
        .transpose(0, 2, 1, 3, 4)
        .reshape(NC, NS, NJ, LANE))
  z2 = jnp.zeros((LANE, EMB), jnp.float32)
  wlin, b_lin = _repack(w_param.T, b_param.T)

  e_w, cnt_w, bg_w = _sc1(xw, wlin, b_lin)

  gw12 = gw1 @ gw2
  gb12r = (gb1 @ gw2).reshape(1, EMB)

  ew_spec = pl.BlockSpec((NC, 1, NJ, LANE, EMB), lambda s: (0, s, 0, 0, 0))
  sw_spec = pl.BlockSpec((NC, 1, NJ, LANE), lambda s: (0, s, 0, 0))

  m_w = pl.pallas_call(
      _tc1_body,
      grid=(NS,),
      in_specs=[ew_spec, sw_spec,
                _const_spec((EMB, EMB)), _const_spec((1, EMB))],
      out_specs=pl.BlockSpec((1, KCH, LANE, EMB), lambda s: (s, 0, 0, 0)),
      out_shape=jax.ShapeDtypeStruct((NS, KCH, LANE, EMB), jnp.float32),
  )(e_w, cnt_w, gw12, gb12r)

  agg_w = _sc2(xw, m_w.reshape(NS * KCH, LANE, EMB), z2)

  out = pl.pallas_call(
      _tc2_body,
      grid=(NS,),
      in_specs=[ew_spec, ew_spec, sw_spec, sw_spec,
                _const_spec((EMB, EMB)), _const_spec((1, EMB)),
                _const_spec((1, EMB)),
                _const_spec((EMB, HID)), _const_spec((1, HID)),
                _const_spec((1, HID)), _const_spec((1, HID)),
                _const_spec((HID, HID)), _const_spec((1, HID)),
                _const_spec((1, HID)), _const_spec((1, HID)),
                _const_spec((HID, 1)), _const_spec((1, 1))],
      out_specs=pl.BlockSpec((1, KCH, LANE), lambda s: (s, 0, 0)),
      out_shape=jax.ShapeDtypeStruct((NS, KCH, LANE), jnp.float32),
  )(e_w, agg_w, cnt_w, bg_w,
    gw12, gb12r, gb2.reshape(1, EMB),
    lw1, lb1.reshape(1, HID), lg1.reshape(1, HID), lbe1.reshape(1, HID),
    lw2, lb2.reshape(1, HID), lg2.reshape(1, HID), lbe2.reshape(1, HID),
    lw3, lb3.reshape(1, 1))
  return out.reshape(-1)


# speedup vs baseline: 1.1061x; 1.1061x over previous
import math

import jax
import jax.numpy as jnp
from jax import lax
from jax.experimental import pallas as pl
from jax.experimental.pallas import tpu as pltpu
from jax.experimental.pallas import tpu_sc as plsc

F = 26
FD = 38461
FN = F * FD
EMB = 16
HID = 64
B = 4096
EPS = 1e-5

NC = 2
NS = 16
FPC = F // NC
LANE = 128
KCH = 2
NJ = FPC * KCH
CTAB = FPC * FD
G2 = 3
R2 = (FPC + G2 - 1) // G2
DINV_S = 1.0 / math.sqrt(27.0)
BNI = 1.0 / math.sqrt(1.0 + EPS)

_MESH = plsc.VectorSubcoreMesh(core_axis_name="c", subcore_axis_name="s")


def _sc1_body(xw_h, w_h, b_h,
              e_o, cnt_o, bg_o,
              xbuf, iloc, iglob, zb, ob, cbuf, bbuf, ebuf, ctab,
              semw, semb, semc):
  c = lax.axis_index("c")
  s = lax.axis_index("s")
  for i in range(LANE // 16):
    zb[pl.ds(i * 16, 16)] = jnp.zeros((16,), jnp.float32)
    ob[pl.ds(i * 16, 16)] = jnp.full((16,), 1.0, jnp.float32)
  pltpu.sync_copy(xw_h.at[c, s], xbuf)
  fbase = (c * FPC) * FD

  @pl.loop(0, NJ)
  def _idx(j):
    jf = j // KCH
    og = fbase + jf * FD
    ol = jf * FD

    @pl.loop(0, LANE, step=16)
    def _v(i):
      v = xbuf[j, pl.ds(i, 16)]
      iglob[j, pl.ds(i, 16)] = v + og
      iloc[j, pl.ds(i, 16)] = v + ol

  wd = [pltpu.async_copy(w_h.at[iglob.at[j]], ebuf.at[j], semw)
        for j in range(NJ)]
  bd = [pltpu.async_copy(b_h.at[iglob.at[j]], bbuf.at[j], semb)
        for j in range(NJ)]

  zd = [pltpu.async_copy(zb, ctab.at[iloc.at[j]], semc) for j in range(NJ)]
  for d in zd:
    d.wait()
  plsc.subcore_barrier()
  ad = [pltpu.async_copy(ob, ctab.at[iloc.at[j]], semc, add=True)
        for j in range(NJ)]
  for d in ad:
    d.wait()
  plsc.subcore_barrier()
  cd = [pltpu.async_copy(ctab.at[iloc.at[j]], cbuf.at[j], semc)
        for j in range(NJ)]
  for d in cd:
    d.wait()
  pltpu.sync_copy(cbuf, cnt_o.at[c, s])
  for d in bd:
    d.wait()
  pltpu.sync_copy(bbuf, bg_o.at[c, s])
  for d in wd:
    d.wait()
  pltpu.sync_copy(ebuf, e_o.at[c, s])


_sc1 = pl.kernel(
    _sc1_body,
    out_type=(
        jax.ShapeDtypeStruct((NC, NS, NJ, LANE, EMB), jnp.float32),
        jax.ShapeDtypeStruct((NC, NS, NJ, LANE), jnp.float32),
        jax.ShapeDtypeStruct((NC, NS, NJ, LANE), jnp.float32),
    ),
    mesh=_MESH,
    scratch_types=[
        pltpu.VMEM((NJ, LANE), jnp.int32),
        pltpu.VMEM((NJ, LANE), jnp.int32),
        pltpu.VMEM((NJ, LANE), jnp.int32),
        pltpu.VMEM((LANE,), jnp.float32),
        pltpu.VMEM((LANE,), jnp.float32),
        pltpu.VMEM((NJ, LANE), jnp.float32),
        pltpu.VMEM((NJ, LANE), jnp.float32),
        pltpu.VMEM((NJ, LANE, EMB), jnp.float32),
        pltpu.VMEM_SHARED((CTAB,), jnp.float32),
        pltpu.SemaphoreType.DMA,
        pltpu.SemaphoreType.DMA,
        pltpu.SemaphoreType.DMA,
    ],
    compiler_params=pltpu.CompilerParams(use_tc_tiling_on_sc=False),
)


def _sc2_body(xw_h, m_h, z2_h,
              agg_o,
              iloc2, mbuf, zrow, gbuf, wtab, sem, semo):
  c = lax.axis_index("c")
  s = lax.axis_index("s")
  pltpu.sync_copy(z2_h, zrow)
  pltpu.sync_copy(xw_h.at[c, s], iloc2)
  pltpu.sync_copy(m_h.at[pl.ds(s * KCH, KCH)], mbuf)

  @pl.loop(0, NJ)
  def _idx(j):
    jf = j // KCH
    ol = (jf % G2) * FD

    @pl.loop(0, LANE, step=16)
    def _v(i):
      iloc2[j, pl.ds(i, 16)] = iloc2[j, pl.ds(i, 16)] + ol

  slot_wr = [None, None]
  for r in range(R2):
    nf = min(G2, FPC - r * G2)
    lo = r * G2 * KCH
    n = nf * KCH
    zd = [pltpu.async_copy(zrow, wtab.at[iloc2.at[lo + t]], sem)
          for t in range(n)]
    for d in zd:
      d.wait()
    plsc.subcore_barrier()
    ad = [pltpu.async_copy(mbuf.at[(lo + t) % KCH], wtab.at[iloc2.at[lo + t]],
                           sem, add=True) for t in range(n)]
    for d in ad:
      d.wait()
    plsc.subcore_barrier()
    for t in range(n):
      sl = t % 2
      if slot_wr[sl] is not None:
        slot_wr[sl].wait()
      pltpu.async_copy(wtab.at[iloc2.at[lo + t]], gbuf.at[sl], sem).wait()
      slot_wr[sl] = pltpu.async_copy(gbuf.at[sl], agg_o.at[c, s, lo + t], semo)
    plsc.subcore_barrier()
  for d in slot_wr:
    if d is not None:
      d.wait()


_sc2 = pl.kernel(
    _sc2_body,
    out_type=jax.ShapeDtypeStruct((NC, NS, NJ, LANE, EMB), jnp.float32),
    mesh=_MESH,
    scratch_types=[
        pltpu.VMEM((NJ, LANE), jnp.int32),
        pltpu.VMEM((KCH, LANE, EMB), jnp.float32),
        pltpu.VMEM((LANE, EMB), jnp.float32),
        pltpu.VMEM((2, LANE, EMB), jnp.float32),
        pltpu.VMEM_SHARED((G2 * FD, EMB), jnp.float32),
        pltpu.SemaphoreType.DMA,
        pltpu.SemaphoreType.DMA,
    ],
    compiler_params=pltpu.CompilerParams(use_tc_tiling_on_sc=False),
)


def _tc1_body(e_ref, cnt_ref, gw12_ref, gb12r_ref, m_ref):
  gw12 = gw12_ref[...]
  gb12r = gb12r_ref[...]
  for k in range(KCH):
    acc = jnp.zeros((LANE, EMB), jnp.float32)
    for c in range(NC):
      for jf in range(FPC):
        j = jf * KCH + k
        df = lax.rsqrt(cnt_ref[c, 0, j] + 1.0)
        acc = acc + df[:, None] * e_ref[c, 0, j]
    m = jnp.dot(acc, gw12, preferred_element_type=jnp.float32)
    m_ref[0, k] = DINV_S * m + gb12r


def _tc2_body(e_ref, agg_ref, cnt_ref, bg_ref,
              gw12_ref, gb12r_ref, gb2r_ref,
              lw1_ref, lb1r_ref, lg1r_ref, lbe1r_ref,
              lw2_ref, lb2r_ref, lg2r_ref, lbe2r_ref,
              lw3_ref, lb3r_ref, out_ref):
  gw12 = gw12_ref[...]
  gb12r = gb12r_ref[...]
  gb2r = gb2r_ref[...]
  for k in range(KCH):
    hsum = jnp.zeros((LANE, EMB), jnp.float32)
    esum = jnp.zeros((LANE, EMB), jnp.float32)
    ssq = jnp.zeros((LANE, EMB), jnp.float32)
    bsum = jnp.zeros((LANE,), jnp.float32)
    for c in range(NC):
      for jf in range(FPC):
        j = jf * KCH + k
        ef = e_ref[c, 0, j]
        af = agg_ref[c, 0, j]
        df = lax.rsqrt(cnt_ref[c, 0, j] + 1.0)[:, None]
        d2 = df * df
        egf = jnp.dot(ef, gw12, preferred_element_type=jnp.float32)
        vf = (d2 * d2) * egf + d2 * gb12r + gb2r
        xf = (DINV_S * df) * af + vf
        hsum = hsum + xf
        esum = esum + ef
        ssq = ssq + ef * xf
        bsum = bsum + bg_ref[c, 0, j]
    fm = 0.5 * jnp.sum(esum * hsum - ssq, axis=1)
    z = jnp.dot(hsum, lw1_ref[...], preferred_element_type=jnp.float32)
    z = jnp.maximum((z + lb1r_ref[...]) * BNI * lg1r_ref[...] + lbe1r_ref[...], 0.0)
    z = jnp.dot(z, lw2_ref[...], preferred_element_type=jnp.float32)
    z = jnp.maximum((z + lb2r_ref[...]) * BNI * lg2r_ref[...] + lbe2r_ref[...], 0.0)
    z = jnp.dot(z, lw3_ref[...], preferred_element_type=jnp.float32) + lb3r_ref[...]
    out_ref[0, k] = bsum + fm + z[:, 0]


def _const_spec(shape):
  return pl.BlockSpec(shape, lambda s: tuple(0 for _ in shape))


def kernel(x, field_mask, new_field_mask, known_mask, b_param, w_param,
           gw1, gb1, gw2, gb2, lw1, lb1, lg1, lbe1, lw2, lb2, lg2, lbe2,
           lw3, lb3):
  x = x.astype(jnp.int32)
  xw = (x.T.reshape(NC, FPC, NS, KCH, LANE)
        .transpose(0, 2, 1, 3, 4)
        .reshape(NC, NS, NJ, LANE))
  z2 = jnp.zeros((LANE, EMB), jnp.float32)
  b_flat = b_param.reshape(-1)

  e_w, cnt_w, bg_w = _sc1(xw, w_param, b_flat)

  gw12 = gw1 @ gw2
  gb12r = (gb1 @ gw2).reshape(1, EMB)

  ew_spec = pl.BlockSpec((NC, 1, NJ, LANE, EMB), lambda s: (0, s, 0, 0, 0))
  sw_spec = pl.BlockSpec((NC, 1, NJ, LANE), lambda s: (0, s, 0, 0))

  m_w = pl.pallas_call(
      _tc1_body,
      grid=(NS,),
      in_specs=[ew_spec, sw_spec,
                _const_spec((EMB, EMB)), _const_spec((1, EMB))],
      out_specs=pl.BlockSpec((1, KCH, LANE, EMB), lambda s: (s, 0, 0, 0)),
      out_shape=jax.ShapeDtypeStruct((NS, KCH, LANE, EMB), jnp.float32),
  )(e_w, cnt_w, gw12, gb12r)

  agg_w = _sc2(xw, m_w.reshape(NS * KCH, LANE, EMB), z2)

  out = pl.pallas_call(
      _tc2_body,
      grid=(NS,),
      in_specs=[ew_spec, ew_spec, sw_spec, sw_spec,
                _const_spec((EMB, EMB)), _const_spec((1, EMB)),
                _const_spec((1, EMB)),
                _const_spec((EMB, HID)), _const_spec((1, HID)),
                _const_spec((1, HID)), _const_spec((1, HID)),
                _const_spec((HID, HID)), _const_spec((1, HID)),
                _const_spec((1, HID)), _const_spec((1, HID)),
                _const_spec((HID, 1)), _const_spec((1, 1))],
      out_specs=pl.BlockSpec((1, KCH, LANE), lambda s: (s, 0, 0)),
      out_shape=jax.ShapeDtypeStruct((NS, KCH, LANE), jnp.float32),
  )(e_w, agg_w, cnt_w, bg_w,
    gw12, gb12r, gb2.reshape(1, EMB),
    lw1, lb1.reshape(1, HID), lg1.reshape(1, HID), lbe1.reshape(1, HID),
    lw2, lb2.reshape(1, HID), lg2.reshape(1, HID), lbe2.reshape(1, HID),
    lw3, lb3.reshape(1, 1))
  return out.reshape(-1)
